# Initial kernel scaffold; baseline (speedup 1.0000x reference)
import jax
import jax.numpy as jnp
from jax.experimental import pallas as pl


def kernel(x, edge_index, W1, b1, W2, b2, W3, b3):
    raise NotImplementedError("write your pallas kernel here")


# speedup vs baseline: 6.1321x; 6.1321x over previous
import functools

import jax
import jax.numpy as jnp
from jax import lax
from jax.experimental import pallas as pl
from jax.experimental.pallas import tpu as pltpu
from jax.experimental.pallas import tpu_sc as plsc

_NC = 2
_NS = 16
_F = 128
_EC = 80
_RPC = 200

_BM = 1000


def _sc_mesh():
    return plsc.VectorSubcoreMesh(
        core_axis_name="c", subcore_axis_name="s",
        num_cores=_NC, num_subcores=_NS)


@functools.lru_cache(maxsize=None)
def _make_spmm(nf, n, e):
    ew = e // _NS
    nchunks = ew // _EC
    nrow_chunks = n // _RPC
    rounds = -(-nrow_chunks // _NS)

    @functools.partial(
        pl.kernel,
        out_type=[jax.ShapeDtypeStruct((n, _F), jnp.float32)
                  for _ in range(nf)],
        mesh=_sc_mesh(),
        scratch_types=[
            pltpu.VMEM((_EC,), jnp.int32),
            pltpu.VMEM((_EC,), jnp.int32),
            pltpu.VMEM((_EC, _F), jnp.float32),
            pltpu.VMEM((_RPC, _F), jnp.float32),
            pltpu.VMEM_SHARED((n, _F), jnp.float32),
            pltpu.SemaphoreType.DMA,
        ],
    )
    def spmm(*refs):
        tables = refs[:nf]
        src_hbm = refs[nf]
        dst_hbm = refs[nf + 1]
        outs = refs[nf + 2:2 * nf + 2]
        isrc, idst, rows, stage, acc, sem = refs[2 * nf + 2:]
        cid = lax.axis_index("c")
        sid = lax.axis_index("s")

        for fc in range(nf):
            tab = tables[fc]
            out = outs[fc]

            @pl.when(cid == (fc % _NC))
            def _(tab=tab, out=out):
                def init_body(r, carry):
                    c = r * _NS + sid

                    @pl.when(c < nrow_chunks)
                    def _():
                        off = c * _RPC
                        pltpu.sync_copy(tab.at[pl.ds(off, _RPC)], stage)
                        pltpu.sync_copy(stage, acc.at[pl.ds(off, _RPC)])
                    return carry
                lax.fori_loop(0, rounds, init_body, 0)
                plsc.subcore_barrier()

                def chunk_body(k, carry):
                    e0 = sid * ew + k * _EC
                    pltpu.sync_copy(src_hbm.at[pl.ds(e0, _EC)], isrc)
                    pltpu.sync_copy(dst_hbm.at[pl.ds(e0, _EC)], idst)
                    pltpu.async_copy(tab.at[isrc], rows, sem).wait()
                    pltpu.sync_copy(rows, acc.at[idst], add=True)
                    return carry
                lax.fori_loop(0, nchunks, chunk_body, 0)
                plsc.subcore_barrier()

                def out_body(r, carry):
                    c = r * _NS + sid

                    @pl.when(c < nrow_chunks)
                    def _():
                        off = c * _RPC
                        pltpu.sync_copy(acc.at[pl.ds(off, _RPC)], stage)
                        pltpu.sync_copy(stage, out.at[pl.ds(off, _RPC)])
                    return carry
                lax.fori_loop(0, rounds, out_body, 0)
                plsc.subcore_barrier()

    return spmm


@functools.lru_cache(maxsize=None)
def _make_prep(n):
    def body(deg_ref, x_ref, dinv_ref, p0a_ref, p0b_ref):
        deg = deg_ref[:, 0:1]
        dinv = lax.rsqrt(deg)
        dinv_ref[...] = dinv
        p0a_ref[...] = x_ref[:, :_F] * dinv
        p0b_ref[...] = x_ref[:, _F:] * dinv

    return pl.pallas_call(
        body,
        grid=(n // _BM,),
        in_specs=[
            pl.BlockSpec((_BM, _F), lambda i: (i, 0)),
            pl.BlockSpec((_BM, 2 * _F), lambda i: (i, 0)),
        ],
        out_specs=[
            pl.BlockSpec((_BM, 1), lambda i: (i, 0)),
            pl.BlockSpec((_BM, _F), lambda i: (i, 0)),
            pl.BlockSpec((_BM, _F), lambda i: (i, 0)),
        ],
        out_shape=[
            jax.ShapeDtypeStruct((n, 1), jnp.float32),
            jax.ShapeDtypeStruct((n, _F), jnp.float32),
            jax.ShapeDtypeStruct((n, _F), jnp.float32),
        ],
    )


@functools.lru_cache(maxsize=None)
def _make_mm(n, nf_in, nf_out, scaled, use_tanh):
    k_dim = nf_in * _F
    n_dim = nf_out * _F

    def body(*refs):
        parts = refs[:nf_in]
        i = nf_in
        if scaled:
            dinv_ref = refs[i]; i += 1
            b_ref = refs[i]; i += 1
        w_ref = refs[i]; i += 1
        outs = refs[i:]
        lhs = jnp.concatenate([p[...] for p in parts], axis=1)
        res = jnp.dot(lhs, w_ref[...],
                      preferred_element_type=jnp.float32,
                      precision=lax.Precision.HIGHEST)
        if scaled:
            dinv = dinv_ref[...]
            res = res * dinv + b_ref[...]
            if use_tanh:
                res = jnp.tanh(res) * dinv
        for j in range(nf_out):
            outs[j][...] = res[:, j * _F:(j + 1) * _F]

    in_specs = [pl.BlockSpec((_BM, _F), lambda i: (i, 0))
                for _ in range(nf_in)]
    if scaled:
        in_specs.append(pl.BlockSpec((_BM, 1), lambda i: (i, 0)))
        in_specs.append(pl.BlockSpec((1, n_dim), lambda i: (0, 0)))
    in_specs.append(pl.BlockSpec((k_dim, n_dim), lambda i: (0, 0)))

    return pl.pallas_call(
        body,
        grid=(n // _BM,),
        in_specs=in_specs,
        out_specs=[pl.BlockSpec((_BM, _F), lambda i: (i, 0))
                   for _ in range(nf_out)],
        out_shape=[jax.ShapeDtypeStruct((n, _F), jnp.float32)
                   for _ in range(nf_out)],
    )


@functools.lru_cache(maxsize=None)
def _make_final(n):
    def body(sa_ref, sb_ref, dinv_ref, b_ref, out_ref):
        s = jnp.concatenate([sa_ref[...], sb_ref[...]], axis=1)
        out_ref[...] = s * dinv_ref[...] + b_ref[...]

    return pl.pallas_call(
        body,
        grid=(n // _BM,),
        in_specs=[
            pl.BlockSpec((_BM, _F), lambda i: (i, 0)),
            pl.BlockSpec((_BM, _F), lambda i: (i, 0)),
            pl.BlockSpec((_BM, 1), lambda i: (i, 0)),
            pl.BlockSpec((1, 2 * _F), lambda i: (0, 0)),
        ],
        out_specs=pl.BlockSpec((_BM, 2 * _F), lambda i: (i, 0)),
        out_shape=jax.ShapeDtypeStruct((n, 2 * _F), jnp.float32),
    )


def kernel(x, edge_index, W1, b1, W2, b2, W3, b3):
    n = x.shape[0]
    e = edge_index.shape[1]
    src = edge_index[0].astype(jnp.int32)
    dst = edge_index[1].astype(jnp.int32)

    ones_tab = jnp.ones((n, _F), dtype=jnp.float32)
    (deg,) = _make_spmm(1, n, e)(ones_tab, src, dst)

    dinv, p0a, p0b = _make_prep(n)(deg, x)

    s0 = _make_spmm(2, n, e)(p0a, p0b, src, dst)
    p1 = _make_mm(n, 2, 4, True, True)(*s0, dinv, b1.reshape(1, -1), W1)
    s1 = _make_spmm(4, n, e)(*p1, src, dst)
    p2 = _make_mm(n, 4, 4, True, True)(*s1, dinv, b2.reshape(1, -1), W2)
    g = _make_mm(n, 4, 2, False, False)(*p2, W3)
    s2 = _make_spmm(2, n, e)(*g, src, dst)
    out = _make_final(n)(*s2, dinv, b3.reshape(1, -1))
    return out
